# baseline (speedup 1.0000x reference)
import functools

import jax
import jax.numpy as jnp
from jax import lax
from jax.experimental import pallas as pl
from jax.experimental.pallas import tpu as pltpu
from jax.experimental.pallas import tpu_sc as plsc

N_BINS = 1024
BATCH = 16384

NC = 2
NS = 16
L = 16
NW = NC * NS
B_PER_W = BATCH // NW

_mesh = plsc.VectorSubcoreMesh(core_axis_name="c", subcore_axis_name="s")


@functools.partial(
    pl.kernel,
    mesh=_mesh,
    out_type=jax.ShapeDtypeStruct((BATCH,), jnp.float32),
    scratch_types=[
        pltpu.VMEM((B_PER_W,), jnp.int32),
        pltpu.VMEM((B_PER_W,), jnp.int32),
        pltpu.VMEM((B_PER_W,), jnp.float32),
        pltpu.SemaphoreType.DMA,
    ],
)
def _gather_kernel(table_hbm, x_hbm, y_hbm, out_hbm, xv, fv, ov, sem):
    wid = lax.axis_index("s") * NC + lax.axis_index("c")
    base = wid * B_PER_W
    pltpu.sync_copy(x_hbm.at[pl.ds(base, B_PER_W)], xv)
    pltpu.sync_copy(y_hbm.at[pl.ds(base, B_PER_W)], fv)

    def step(i, carry):
        sl = pl.ds(i * L, L)
        fv[sl] = xv[sl] * N_BINS + fv[sl]
        return carry

    lax.fori_loop(0, B_PER_W // L, step, 0)

    pltpu.async_copy(table_hbm.at[fv], ov, sem).wait()
    pltpu.sync_copy(ov, out_hbm.at[pl.ds(base, B_PER_W)])


def kernel(energies, x_indices, y_indices):
    flat = energies.reshape(-1)
    return _gather_kernel(flat, x_indices, y_indices)


# speedup vs baseline: 1.1083x; 1.1083x over previous
import functools

import jax
import jax.numpy as jnp
from jax import lax
from jax.experimental import pallas as pl
from jax.experimental.pallas import tpu as pltpu
from jax.experimental.pallas import tpu_sc as plsc

N_BINS = 1024
BATCH = 16384

NC = 2
NS = 16
L = 16
NW = NC * NS
B_PER_W = BATCH // NW


_mesh = plsc.VectorSubcoreMesh(core_axis_name="c", subcore_axis_name="s")


@functools.partial(
    pl.kernel,
    mesh=_mesh,
    out_type=jax.ShapeDtypeStruct((BATCH,), jnp.float32),
    scratch_types=[
        pltpu.VMEM((B_PER_W,), jnp.int32),
        pltpu.VMEM((B_PER_W,), jnp.int32),
        pltpu.VMEM((B_PER_W,), jnp.float32),
        pltpu.SemaphoreType.DMA,
    ],
)
def _gather_kernel(table_hbm, x_hbm, y_hbm, out_hbm, xv, fv, ov, sem):
    wid = lax.axis_index("s") * NC + lax.axis_index("c")
    base = wid * B_PER_W
    pltpu.sync_copy(x_hbm.at[pl.ds(base, B_PER_W)], xv)
    pltpu.sync_copy(y_hbm.at[pl.ds(base, B_PER_W)], fv)

    def idx_step(i, carry):
        sl = pl.ds(i * L, L)
        x = xv[sl]
        y = fv[sl]
        fv[sl] = (
            ((x >> 3) << 13) + ((y >> 7) << 10) + ((x & 7) << 7) + (y & 127)
        )
        return carry

    lax.fori_loop(0, B_PER_W // L, idx_step, 0)

    pltpu.async_copy(table_hbm.at[fv], ov, sem).wait()
    pltpu.sync_copy(ov, out_hbm.at[pl.ds(base, B_PER_W)])


def kernel(energies, x_indices, y_indices):
    tiled_view = (
        energies.reshape(N_BINS // 8, 8, N_BINS // 128, 128)
        .transpose(0, 2, 1, 3)
        .reshape(N_BINS * N_BINS)
    )
    return _gather_kernel(tiled_view, x_indices, y_indices)


# speedup vs baseline: 1.1359x; 1.0248x over previous
import functools

import jax
import jax.numpy as jnp
from jax import lax
from jax.experimental import pallas as pl
from jax.experimental.pallas import tpu as pltpu
from jax.experimental.pallas import tpu_sc as plsc

N_BINS = 1024
BATCH = 16384

NC = 2
NS = 16
L = 16
NW = NC * NS
B_PER_W = BATCH // NW


_mesh = plsc.VectorSubcoreMesh(core_axis_name="c", subcore_axis_name="s")


@functools.partial(
    pl.kernel,
    mesh=_mesh,
    out_type=jax.ShapeDtypeStruct((BATCH,), jnp.float32),
    scratch_types=[
        pltpu.VMEM((B_PER_W,), jnp.int32),
        pltpu.VMEM((B_PER_W,), jnp.int32),
        pltpu.VMEM((B_PER_W,), jnp.float32),
        pltpu.SemaphoreType.DMA,
        pltpu.SemaphoreType.DMA,
    ],
)
def _gather_kernel(table_hbm, x_hbm, y_hbm, out_hbm, xv, fv, ov, sem_x, sem_y):
    wid = lax.axis_index("s") * NC + lax.axis_index("c")
    base = wid * B_PER_W
    cp_x = pltpu.async_copy(x_hbm.at[pl.ds(base, B_PER_W)], xv, sem_x)
    cp_y = pltpu.async_copy(y_hbm.at[pl.ds(base, B_PER_W)], fv, sem_y)
    cp_x.wait()
    cp_y.wait()

    @plsc.parallel_loop(0, B_PER_W, step=L, unroll=8)
    def idx_step(i):
        sl = pl.ds(i, L)
        x = xv[sl]
        y = fv[sl]
        fv[sl] = (
            ((x >> 3) << 13) + ((y >> 7) << 10) + ((x & 7) << 7) + (y & 127)
        )

    pltpu.async_copy(table_hbm.at[fv], ov, sem_x).wait()
    pltpu.sync_copy(ov, out_hbm.at[pl.ds(base, B_PER_W)])


def kernel(energies, x_indices, y_indices):
    tiled_view = (
        energies.reshape(N_BINS // 8, 8, N_BINS // 128, 128)
        .transpose(0, 2, 1, 3)
        .reshape(N_BINS * N_BINS)
    )
    return _gather_kernel(tiled_view, x_indices, y_indices)


# speedup vs baseline: 1.1493x; 1.0118x over previous
import functools

import jax
import jax.numpy as jnp
from jax import lax
from jax.experimental import pallas as pl
from jax.experimental.pallas import tpu as pltpu
from jax.experimental.pallas import tpu_sc as plsc

N_BINS = 1024
BATCH = 16384

NC = 2
NS = 16
L = 16
NW = NC * NS
B_PER_W = BATCH // NW


_mesh = plsc.VectorSubcoreMesh(core_axis_name="c", subcore_axis_name="s")


@functools.partial(
    pl.kernel,
    mesh=_mesh,
    out_type=jax.ShapeDtypeStruct((BATCH,), jnp.float32),
    scratch_types=[
        pltpu.VMEM((B_PER_W,), jnp.int32),
        pltpu.VMEM((B_PER_W,), jnp.int32),
        pltpu.VMEM((B_PER_W,), jnp.float32),
        pltpu.SemaphoreType.DMA,
        pltpu.SemaphoreType.DMA,
        pltpu.SemaphoreType.DMA,
        pltpu.SemaphoreType.DMA,
    ],
)
def _gather_kernel(table_hbm, x_hbm, y_hbm, out_hbm, xv, fv, ov, s0, s1, g0, g1):
    wid = lax.axis_index("s") * NC + lax.axis_index("c")
    base = wid * B_PER_W
    H = B_PER_W // 2
    cx0 = pltpu.async_copy(x_hbm.at[pl.ds(base, H)], xv.at[pl.ds(0, H)], s0)
    cy0 = pltpu.async_copy(y_hbm.at[pl.ds(base, H)], fv.at[pl.ds(0, H)], s0)
    cx1 = pltpu.async_copy(x_hbm.at[pl.ds(base + H, H)], xv.at[pl.ds(H, H)], s1)
    cy1 = pltpu.async_copy(y_hbm.at[pl.ds(base + H, H)], fv.at[pl.ds(H, H)], s1)
    cx0.wait()
    cy0.wait()

    @plsc.parallel_loop(0, H, step=L, unroll=8)
    def idx_step0(i):
        sl = pl.ds(i, L)
        x = xv[sl]
        y = fv[sl]
        fv[sl] = (
            ((x >> 3) << 13) + ((y >> 7) << 10) + ((x & 7) << 7) + (y & 127)
        )

    gth0 = pltpu.async_copy(
        table_hbm.at[fv.at[pl.ds(0, H)]], ov.at[pl.ds(0, H)], g0
    )
    cx1.wait()
    cy1.wait()

    @plsc.parallel_loop(H, B_PER_W, step=L, unroll=8)
    def idx_step1(i):
        sl = pl.ds(i, L)
        x = xv[sl]
        y = fv[sl]
        fv[sl] = (
            ((x >> 3) << 13) + ((y >> 7) << 10) + ((x & 7) << 7) + (y & 127)
        )

    gth1 = pltpu.async_copy(
        table_hbm.at[fv.at[pl.ds(H, H)]], ov.at[pl.ds(H, H)], g1
    )
    gth0.wait()
    wb0 = pltpu.async_copy(ov.at[pl.ds(0, H)], out_hbm.at[pl.ds(base, H)], s0)
    gth1.wait()
    wb1 = pltpu.async_copy(
        ov.at[pl.ds(H, H)], out_hbm.at[pl.ds(base + H, H)], s1
    )
    wb0.wait()
    wb1.wait()


def kernel(energies, x_indices, y_indices):
    tiled_view = (
        energies.reshape(N_BINS // 8, 8, N_BINS // 128, 128)
        .transpose(0, 2, 1, 3)
        .reshape(N_BINS * N_BINS)
    )
    return _gather_kernel(tiled_view, x_indices, y_indices)
